# Initial kernel scaffold; baseline (speedup 1.0000x reference)
import jax
import jax.numpy as jnp
from jax.experimental import pallas as pl


def kernel(points, params):
    raise NotImplementedError("write your pallas kernel here")


# speedup vs baseline: 14.0339x; 14.0339x over previous
import functools
import math

import jax
import jax.numpy as jnp
import numpy as np
from jax import lax
from jax.experimental import pallas as pl
from jax.experimental.pallas import tpu as pltpu
from jax.experimental.pallas import tpu_sc as plsc

_B = 8
_N = 1024
_DENOM = np.float32(np.sqrt(np.float32(1.0 + 1e-5)))
_NC = 2
_NS = 16
_NW = _NC * _NS


def _fps_body(pts_ref, ox_ref, oy_ref, oz_ref, *, S):
    x = pts_ref[:, 0, :]
    y = pts_ref[:, 1, :]
    z = pts_ref[:, 2, :]
    n = x.shape[1]
    lane = lax.broadcasted_iota(jnp.int32, (_B, n), 1)
    slot = lax.broadcasted_iota(jnp.int32, (_B, S), 1)

    def step(t, carry):
        dists, far, ox, oy, oz = carry
        oh = (lane == far).astype(jnp.float32)
        cx = jnp.sum(oh * x, axis=1, keepdims=True)
        cy = jnp.sum(oh * y, axis=1, keepdims=True)
        cz = jnp.sum(oh * z, axis=1, keepdims=True)
        sl = (slot == t).astype(jnp.float32)
        ox = ox + cx * sl
        oy = oy + cy * sl
        oz = oz + cz * sl
        d = (x - cx) ** 2 + (y - cy) ** 2 + (z - cz) ** 2
        dists = jnp.minimum(dists, d)
        m = jnp.max(dists, axis=1, keepdims=True)
        nxt = jnp.min(jnp.where(dists == m, lane, n), axis=1, keepdims=True)
        return dists, nxt.astype(jnp.int32), ox, oy, oz

    big = jnp.full((_B, n), 1e10, jnp.float32)
    zero = jnp.zeros((_B, S), jnp.float32)
    _, _, ox, oy, oz = lax.fori_loop(
        0, S, step, (big, jnp.zeros((_B, 1), jnp.int32), zero, zero, zero))
    ox_ref[...] = ox
    oy_ref[...] = oy
    oz_ref[...] = oz


def _fps(pts, S):
    return pl.pallas_call(
        functools.partial(_fps_body, S=S),
        out_shape=[jax.ShapeDtypeStruct((_B, S), jnp.float32)] * 3,
    )(pts)


def _d2_body(c_ref, p_ref, o_ref):
    c = c_ref[0]
    dx = c[:, 0:1] - p_ref[0, 0:1, :]
    dy = c[:, 1:2] - p_ref[0, 1:2, :]
    dz = c[:, 2:3] - p_ref[0, 2:3, :]
    o_ref[0] = dx * dx + dy * dy + dz * dz


def _pair_d2(c, pts, S, N):
    return pl.pallas_call(
        _d2_body,
        grid=(_B,),
        in_specs=[pl.BlockSpec((1, S, 3), lambda b: (b, 0, 0)),
                  pl.BlockSpec((1, 3, N), lambda b: (b, 0, 0))],
        out_specs=pl.BlockSpec((1, S, N), lambda b: (b, 0, 0)),
        out_shape=jax.ShapeDtypeStruct((_B, S, N), jnp.float32),
    )(c, pts)


def _proj_body(p_ref, c_ref, w1, w2, w3, x1, x2, x3, u1, u2, u3, v1, v2, v3):
    p = p_ref[0]
    c = c_ref[0]
    for w, wx, u, v in ((w1, x1, u1, v1), (w2, x2, u2, v2), (w3, x3, u3, v3)):
        u[0] = jnp.dot(p, w[...], preferred_element_type=jnp.float32)
        v[0] = jnp.dot(c, wx[...], preferred_element_type=jnp.float32)


def _proj(P, c, scale_params, N, S):
    Cin = P.shape[-1]
    wTs = [jnp.pad(jnp.transpose(l[0]["W"]),
                   ((0, 0), (0, 128 - l[0]["W"].shape[0])))
           for l in scale_params]
    wxTs = [w[0:3] for w in wTs]
    C1s = [128 for _ in wTs]
    full = lambda s: pl.BlockSpec(s, lambda b: tuple(0 for _ in s))
    in_specs = [pl.BlockSpec((1, N, Cin), lambda b: (b, 0, 0)),
                pl.BlockSpec((1, S, 3), lambda b: (b, 0, 0))]
    in_specs += [full(w.shape) for w in wTs] + [full(w.shape) for w in wxTs]
    out_specs = ([pl.BlockSpec((1, N, c1), lambda b: (b, 0, 0)) for c1 in C1s]
                 + [pl.BlockSpec((1, S, c1), lambda b: (b, 0, 0)) for c1 in C1s])
    out_shape = ([jax.ShapeDtypeStruct((_B, N, c1), jnp.float32) for c1 in C1s]
                 + [jax.ShapeDtypeStruct((_B, S, c1), jnp.float32) for c1 in C1s])
    outs = pl.pallas_call(
        _proj_body, grid=(_B,), in_specs=in_specs, out_specs=out_specs,
        out_shape=out_shape,
    )(P, c, *wTs, *wxTs)
    return outs[:3], outs[3:]


def _ballq_body(d2_ref, o_ref, *, R2, K, S, N):
    d2 = d2_ref[0]
    b = pl.program_id(0)
    lane = lax.broadcasted_iota(jnp.int32, (S, N), 1)
    slot = lax.broadcasted_iota(jnp.int32, (S, K), 1)
    key0 = jnp.where(d2 < R2, lane, N)

    def step(r, carry):
        key, acc = carry
        m = jnp.min(key, axis=1, keepdims=True)
        acc = acc + m * (slot == r).astype(jnp.int32)
        key = jnp.where(key == m, N, key)
        return key, acc

    _, idx = lax.fori_loop(0, K, step,
                           (key0, jnp.zeros((S, K), jnp.int32)))
    first = idx[:, 0:1]
    first = jnp.where(first == N, 0, first)
    idx = jnp.where(idx == N, first, idx)
    o_ref[0] = idx + b * N


def _ballq_idx(d2, rr, K, S, N):
    return pl.pallas_call(
        functools.partial(_ballq_body, R2=rr, K=K, S=S, N=N),
        grid=(_B,),
        in_specs=[pl.BlockSpec((1, S, N), lambda b: (b, 0, 0))],
        out_specs=pl.BlockSpec((1, S, K), lambda b: (b, 0, 0)),
        out_shape=jax.ShapeDtypeStruct((_B, S, K), jnp.int32),
    )(d2)


def _sc_gather_body(idx_ref, tab_ref, out_ref, idxv, rows, sem, *,
                    K, RPW, R):
    wid = lax.axis_index("s") * _NC + lax.axis_index("c")

    def chunk(ci, _):
        base = wid * RPW + ci * R
        pltpu.sync_copy(idx_ref.at[pl.ds(base, R)], idxv)
        cps = [pltpu.async_copy(tab_ref.at[idxv.at[r]], rows.at[r], sem)
               for r in range(R)]
        for cp in cps:
            cp.wait()
        pltpu.sync_copy(rows, out_ref.at[pl.ds(base, R)])
        return 0

    lax.fori_loop(0, RPW // R, chunk, 0)


def _ball_gather(idxall, tab, K, S):
    C1 = tab.shape[-1]
    RPW = (_B * S) // _NW
    R = 8
    while R > 1 and (R * (K * C1 + K) * 4 > 380_000 or RPW % R != 0):
        R //= 2
    body = functools.partial(_sc_gather_body, K=K, RPW=RPW, R=R)
    f = pl.kernel(
        body,
        out_type=jax.ShapeDtypeStruct((_B * S, K, C1), jnp.float32),
        mesh=plsc.VectorSubcoreMesh(core_axis_name="c", subcore_axis_name="s"),
        scratch_types=[
            pltpu.VMEM((R, K), jnp.int32),
            pltpu.VMEM((R, K, C1), jnp.float32),
            pltpu.SemaphoreType.DMA,
        ],
    )
    return f(idxall, tab)


def _mlp_body(g_ref, v_ref, *refs, raw, K, SC):
    rs = list(refs)
    out_ref = rs.pop()
    g = g_ref[0]
    v = v_ref[0]
    x = (g - v[:, None, :]).reshape(SC * K, g.shape[-1])
    if raw:
        x = jnp.dot(x, rs.pop(0)[...], preferred_element_type=jnp.float32)
    for li in range(3):
        if li > 0:
            x = jnp.dot(x, rs.pop(0)[...], preferred_element_type=jnp.float32)
        gam = rs.pop(0)[...]
        bet = rs.pop(0)[...]
        x = jnp.maximum(x / _DENOM * gam + bet, 0.0)
    out_ref[0] = jnp.max(x.reshape(SC, K, x.shape[-1]), axis=1)


def _mlp(g, v, layers, K, S, raw):
    C1g = g.shape[-1]
    C1 = layers[0]["W"].shape[0]
    C3 = layers[2]["W"].shape[0]
    SC = min(S, 8192 // K)
    ws = []
    if raw:
        w1T = jnp.transpose(layers[0]["W"])
        ws.append(jnp.pad(w1T, ((0, C1g - 3), (0, 0))))
        pad1 = 0
    else:
        pad1 = C1g - C1
    for li in range(3):
        if li > 0:
            w = jnp.transpose(layers[li]["W"])
            if li == 1 and pad1:
                w = jnp.pad(w, ((0, pad1), (0, 0)))
            ws.append(w)
        gam = layers[li]["gamma"].reshape(1, -1)
        bet = layers[li]["beta"].reshape(1, -1)
        if li == 0 and pad1:
            gam = jnp.pad(gam, ((0, 0), (0, pad1)))
            bet = jnp.pad(bet, ((0, 0), (0, pad1)))
        ws.append(gam)
        ws.append(bet)
    full = lambda s: pl.BlockSpec(s, lambda b, sb: tuple(0 for _ in s))
    in_specs = [pl.BlockSpec((1, SC, K, C1g), lambda b, sb: (b, sb, 0, 0)),
                pl.BlockSpec((1, SC, C1g), lambda b, sb: (b, sb, 0))]
    in_specs += [full(w.shape) for w in ws]
    body = functools.partial(_mlp_body, raw=raw, K=K, SC=SC)
    return pl.pallas_call(
        body, grid=(_B, S // SC), in_specs=in_specs,
        out_specs=pl.BlockSpec((1, SC, C3), lambda b, sb: (b, sb, 0)),
        out_shape=jax.ShapeDtypeStruct((_B, S, C3), jnp.float32),
    )(g, v, *ws)


def _head_body(p_ref, *refs):
    rs = list(refs)
    out_ref = rs.pop()
    x = p_ref[0]
    for _ in range(3):
        x = jnp.dot(x, rs.pop(0)[...], preferred_element_type=jnp.float32)
        x = jnp.maximum(x / _DENOM * rs.pop(0)[...] + rs.pop(0)[...], 0.0)
    x = jnp.max(x, axis=0, keepdims=True)
    for _ in range(2):
        x = jnp.dot(x, rs.pop(0)[...], preferred_element_type=jnp.float32)
        x = jnp.maximum(x / _DENOM * rs.pop(0)[...] + rs.pop(0)[...], 0.0)
    out_ref[0] = jnp.dot(x, rs.pop(0)[...],
                         preferred_element_type=jnp.float32) + rs.pop(0)[...]


def _head(P, sa3, glob, clf, S):
    Cin = P.shape[-1]
    ws = []
    for lyr in sa3 + glob:
        ws += [jnp.transpose(lyr["W"]), lyr["gamma"].reshape(1, -1),
               lyr["beta"].reshape(1, -1)]
    ws += [jnp.transpose(clf["W"]), clf["b"].reshape(1, -1)]
    full = lambda s: pl.BlockSpec(s, lambda b: tuple(0 for _ in s))
    in_specs = [pl.BlockSpec((1, S, Cin), lambda b: (b, 0, 0))]
    in_specs += [full(w.shape) for w in ws]
    return pl.pallas_call(
        _head_body, grid=(_B,), in_specs=in_specs,
        out_specs=pl.BlockSpec((1, 1, 40), lambda b: (b, 0, 0)),
        out_shape=jax.ShapeDtypeStruct((_B, 1, 40), jnp.float32),
    )(P, *ws).reshape(_B, 40)


_SA1 = ((0.1, 16), (0.2, 32), (0.4, 128))
_SA2 = ((0.2, 32), (0.4, 64), (0.8, 128))


def _sa_msg_raw(pts, S, N, scale_cfg, scale_params):
    ox, oy, oz = _fps(pts, S)
    c = jnp.stack([ox, oy, oz], axis=-1)
    d2 = _pair_d2(c, pts, S, N)
    tab = jnp.pad(jnp.transpose(pts, (0, 2, 1)),
                  ((0, 0), (0, 0), (0, 125))).reshape(_B * N, 128)
    cpad = jnp.pad(c, ((0, 0), (0, 0), (0, 125)))
    feats = []
    for (radius, K), layers in zip(scale_cfg, scale_params):
        idx = _ballq_idx(d2, radius * radius, K, S, N).reshape(_B * S, K)
        g = _ball_gather(idx, tab, K, S)
        f = _mlp(g.reshape(_B, S, K, 128), cpad, layers, K, S, raw=True)
        feats.append(f)
    return c, jnp.concatenate(feats, axis=-1)


def _sa_msg_proj(c_prev, f_prev, S, N, scale_cfg, scale_params):
    pts = jnp.transpose(c_prev, (0, 2, 1))
    ox, oy, oz = _fps(pts, S)
    c = jnp.stack([ox, oy, oz], axis=-1)
    d2 = _pair_d2(c, pts, S, N)
    P = jnp.concatenate([c_prev, f_prev], axis=-1)
    us, vs = _proj(P, c, scale_params, N, S)
    feats = []
    for (radius, K), layers, u, v in zip(scale_cfg, scale_params, us, vs):
        idx = _ballq_idx(d2, radius * radius, K, S, N).reshape(_B * S, K)
        g = _ball_gather(idx, u.reshape(_B * N, 128), K, S)
        f = _mlp(g.reshape(_B, S, K, 128), v, layers, K, S, raw=False)
        feats.append(f)
    return c, jnp.concatenate(feats, axis=-1)


def kernel(points, params):
    c1, f1 = _sa_msg_raw(points[:, 0:3, :], 512, _N, _SA1, params["sa1"])
    c2, f2 = _sa_msg_proj(c1, f1, 128, 512, _SA2, params["sa2"])
    P3 = jnp.concatenate([c2, f2], axis=-1)
    cls = _head(P3, params["sa3"], params["mlp_global"], params["classifier"],
                128)
    kpi = jnp.zeros((_B, 1024), jnp.int32)
    return cls, kpi


# speedup vs baseline: 14.9262x; 1.0636x over previous
import functools
import math

import jax
import jax.numpy as jnp
import numpy as np
from jax import lax
from jax.experimental import pallas as pl
from jax.experimental.pallas import tpu as pltpu
from jax.experimental.pallas import tpu_sc as plsc

_B = 8
_N = 1024
_DENOM = np.float32(np.sqrt(np.float32(1.0 + 1e-5)))
_NC = 2
_NS = 16
_NW = _NC * _NS


def _fps_body(pts_ref, ox_ref, oy_ref, oz_ref, *, S):
    x = pts_ref[:, 0, :]
    y = pts_ref[:, 1, :]
    z = pts_ref[:, 2, :]
    n = x.shape[1]
    lane = lax.broadcasted_iota(jnp.int32, (_B, n), 1)
    slot = lax.broadcasted_iota(jnp.int32, (_B, S), 1)

    def step(t, carry):
        dists, far, ox, oy, oz = carry
        oh = (lane == far).astype(jnp.float32)
        cx = jnp.sum(oh * x, axis=1, keepdims=True)
        cy = jnp.sum(oh * y, axis=1, keepdims=True)
        cz = jnp.sum(oh * z, axis=1, keepdims=True)
        sl = (slot == t).astype(jnp.float32)
        ox = ox + cx * sl
        oy = oy + cy * sl
        oz = oz + cz * sl
        d = (x - cx) ** 2 + (y - cy) ** 2 + (z - cz) ** 2
        dists = jnp.minimum(dists, d)
        m = jnp.max(dists, axis=1, keepdims=True)
        nxt = jnp.min(jnp.where(dists == m, lane, n), axis=1, keepdims=True)
        return dists, nxt.astype(jnp.int32), ox, oy, oz

    big = jnp.full((_B, n), 1e10, jnp.float32)
    zero = jnp.zeros((_B, S), jnp.float32)
    _, _, ox, oy, oz = lax.fori_loop(
        0, S, step, (big, jnp.zeros((_B, 1), jnp.int32), zero, zero, zero))
    ox_ref[...] = ox
    oy_ref[...] = oy
    oz_ref[...] = oz


def _fps(pts, S):
    return pl.pallas_call(
        functools.partial(_fps_body, S=S),
        out_shape=[jax.ShapeDtypeStruct((_B, S), jnp.float32)] * 3,
    )(pts)


def _d2_body(c_ref, p_ref, o_ref):
    c = c_ref[0]
    dx = c[:, 0:1] - p_ref[0, 0:1, :]
    dy = c[:, 1:2] - p_ref[0, 1:2, :]
    dz = c[:, 2:3] - p_ref[0, 2:3, :]
    o_ref[0] = dx * dx + dy * dy + dz * dz


def _pair_d2(c, pts, S, N):
    return pl.pallas_call(
        _d2_body,
        grid=(_B,),
        in_specs=[pl.BlockSpec((1, S, 3), lambda b: (b, 0, 0)),
                  pl.BlockSpec((1, 3, N), lambda b: (b, 0, 0))],
        out_specs=pl.BlockSpec((1, S, N), lambda b: (b, 0, 0)),
        out_shape=jax.ShapeDtypeStruct((_B, S, N), jnp.float32),
    )(c, pts)


def _proj_body(p_ref, c_ref, w1, w2, w3, x1, x2, x3, u1, u2, u3, v1, v2, v3):
    p = p_ref[0]
    c = c_ref[0]
    for w, wx, u, v in ((w1, x1, u1, v1), (w2, x2, u2, v2), (w3, x3, u3, v3)):
        u[0] = jnp.dot(p, w[...], preferred_element_type=jnp.float32)
        v[0] = jnp.dot(c, wx[...], preferred_element_type=jnp.float32)


def _proj(P, c, scale_params, N, S):
    Cin = P.shape[-1]
    wTs = [jnp.pad(jnp.transpose(l[0]["W"]),
                   ((0, 0), (0, 128 - l[0]["W"].shape[0])))
           for l in scale_params]
    wxTs = [w[0:3] for w in wTs]
    C1s = [128 for _ in wTs]
    full = lambda s: pl.BlockSpec(s, lambda b: tuple(0 for _ in s))
    in_specs = [pl.BlockSpec((1, N, Cin), lambda b: (b, 0, 0)),
                pl.BlockSpec((1, S, 3), lambda b: (b, 0, 0))]
    in_specs += [full(w.shape) for w in wTs] + [full(w.shape) for w in wxTs]
    out_specs = ([pl.BlockSpec((1, N, c1), lambda b: (b, 0, 0)) for c1 in C1s]
                 + [pl.BlockSpec((1, S, c1), lambda b: (b, 0, 0)) for c1 in C1s])
    out_shape = ([jax.ShapeDtypeStruct((_B, N, c1), jnp.float32) for c1 in C1s]
                 + [jax.ShapeDtypeStruct((_B, S, c1), jnp.float32) for c1 in C1s])
    outs = pl.pallas_call(
        _proj_body, grid=(_B,), in_specs=in_specs, out_specs=out_specs,
        out_shape=out_shape,
    )(P, c, *wTs, *wxTs)
    return outs[:3], outs[3:]


def _ballq_body(d2_ref, o_ref, *, R2, K, S, N):
    d2 = d2_ref[0]
    b = pl.program_id(0)
    lane = lax.broadcasted_iota(jnp.int32, (S, N), 1)
    slot = lax.broadcasted_iota(jnp.int32, (S, K), 1)
    key0 = jnp.where(d2 < R2, lane, N)

    def step(r, carry):
        key, acc = carry
        m = jnp.min(key, axis=1, keepdims=True)
        acc = acc + m * (slot == r).astype(jnp.int32)
        key = jnp.where(key == m, N, key)
        return key, acc

    _, idx = lax.fori_loop(0, K, step,
                           (key0, jnp.zeros((S, K), jnp.int32)))
    first = idx[:, 0:1]
    first = jnp.where(first == N, 0, first)
    idx = jnp.where(idx == N, first, idx)
    o_ref[0] = idx + b * N


def _ballq_idx(d2, rr, K, S, N):
    return pl.pallas_call(
        functools.partial(_ballq_body, R2=rr, K=K, S=S, N=N),
        grid=(_B,),
        in_specs=[pl.BlockSpec((1, S, N), lambda b: (b, 0, 0))],
        out_specs=pl.BlockSpec((1, S, K), lambda b: (b, 0, 0)),
        out_shape=jax.ShapeDtypeStruct((_B, S, K), jnp.int32),
    )(d2)


def _sc_gather_body(idx_ref, tab_ref, out_ref, idxv0, rows0, idxv1, rows1,
                    sem0, sem1, *, K, RPW, R):
    wid = lax.axis_index("s") * _NC + lax.axis_index("c")
    nch = RPW // R
    bufs = ((idxv0, rows0, sem0), (idxv1, rows1, sem1))

    def fire(c, buf):
        idxv, rows, sem = buf
        pltpu.sync_copy(idx_ref.at[pl.ds(wid * RPW + c * R, R)], idxv)
        for r in range(R):
            pltpu.async_copy(tab_ref.at[idxv.at[r]], rows.at[r], sem)

    def drain_write(c, buf):
        idxv, rows, sem = buf
        for r in range(R):
            pltpu.make_async_copy(tab_ref.at[idxv.at[r]], rows.at[r],
                                  sem).wait()
        pltpu.sync_copy(rows, out_ref.at[pl.ds(wid * RPW + c * R, R)])

    fire(0, bufs[0])

    def pipe(p, _):
        c = 2 * p
        fire(c + 1, bufs[1])
        drain_write(c, bufs[0])
        fire(c + 2, bufs[0])
        drain_write(c + 1, bufs[1])
        return 0

    lax.fori_loop(0, nch // 2 - 1, pipe, 0)
    c = nch - 2
    fire(c + 1, bufs[1])
    drain_write(c, bufs[0])
    drain_write(c + 1, bufs[1])


def _ball_gather(idxall, tab, K, S):
    C1 = tab.shape[-1]
    RPW = (_B * S) // _NW
    R = 16
    while R > 1 and (2 * R * (K * C1 + 2 * K) * 4 > 380_000
                     or RPW % (2 * R) != 0):
        R //= 2
    body = functools.partial(_sc_gather_body, K=K, RPW=RPW, R=R)
    f = pl.kernel(
        body,
        out_type=jax.ShapeDtypeStruct((_B * S, K, C1), jnp.float32),
        mesh=plsc.VectorSubcoreMesh(core_axis_name="c", subcore_axis_name="s"),
        scratch_types=[
            pltpu.VMEM((R, K), jnp.int32),
            pltpu.VMEM((R, K, C1), jnp.float32),
            pltpu.VMEM((R, K), jnp.int32),
            pltpu.VMEM((R, K, C1), jnp.float32),
            pltpu.SemaphoreType.DMA,
            pltpu.SemaphoreType.DMA,
        ],
    )
    return f(idxall, tab)


def _mlp_body(g_ref, v_ref, *refs, raw, K, SC):
    rs = list(refs)
    out_ref = rs.pop()
    g = g_ref[0]
    v = v_ref[0]
    x = (g - v[:, None, :]).reshape(SC * K, g.shape[-1])
    if raw:
        x = jnp.dot(x, rs.pop(0)[...], preferred_element_type=jnp.float32)
    for li in range(3):
        if li > 0:
            x = jnp.dot(x, rs.pop(0)[...], preferred_element_type=jnp.float32)
        gam = rs.pop(0)[...]
        bet = rs.pop(0)[...]
        x = jnp.maximum(x / _DENOM * gam + bet, 0.0)
    out_ref[0] = jnp.max(x.reshape(SC, K, x.shape[-1]), axis=1)


def _mlp(g, v, layers, K, S, raw):
    C1g = g.shape[-1]
    C1 = layers[0]["W"].shape[0]
    C3 = layers[2]["W"].shape[0]
    SC = min(S, 8192 // K)
    ws = []
    if raw:
        w1T = jnp.transpose(layers[0]["W"])
        ws.append(jnp.pad(w1T, ((0, C1g - 3), (0, 0))))
        pad1 = 0
    else:
        pad1 = C1g - C1
    for li in range(3):
        if li > 0:
            w = jnp.transpose(layers[li]["W"])
            if li == 1 and pad1:
                w = jnp.pad(w, ((0, pad1), (0, 0)))
            ws.append(w)
        gam = layers[li]["gamma"].reshape(1, -1)
        bet = layers[li]["beta"].reshape(1, -1)
        if li == 0 and pad1:
            gam = jnp.pad(gam, ((0, 0), (0, pad1)))
            bet = jnp.pad(bet, ((0, 0), (0, pad1)))
        ws.append(gam)
        ws.append(bet)
    full = lambda s: pl.BlockSpec(s, lambda b, sb: tuple(0 for _ in s))
    in_specs = [pl.BlockSpec((1, SC, K, C1g), lambda b, sb: (b, sb, 0, 0)),
                pl.BlockSpec((1, SC, C1g), lambda b, sb: (b, sb, 0))]
    in_specs += [full(w.shape) for w in ws]
    body = functools.partial(_mlp_body, raw=raw, K=K, SC=SC)
    return pl.pallas_call(
        body, grid=(_B, S // SC), in_specs=in_specs,
        out_specs=pl.BlockSpec((1, SC, C3), lambda b, sb: (b, sb, 0)),
        out_shape=jax.ShapeDtypeStruct((_B, S, C3), jnp.float32),
    )(g, v, *ws)


def _head_body(p_ref, *refs):
    rs = list(refs)
    out_ref = rs.pop()
    x = p_ref[0]
    for _ in range(3):
        x = jnp.dot(x, rs.pop(0)[...], preferred_element_type=jnp.float32)
        x = jnp.maximum(x / _DENOM * rs.pop(0)[...] + rs.pop(0)[...], 0.0)
    x = jnp.max(x, axis=0, keepdims=True)
    for _ in range(2):
        x = jnp.dot(x, rs.pop(0)[...], preferred_element_type=jnp.float32)
        x = jnp.maximum(x / _DENOM * rs.pop(0)[...] + rs.pop(0)[...], 0.0)
    out_ref[0] = jnp.dot(x, rs.pop(0)[...],
                         preferred_element_type=jnp.float32) + rs.pop(0)[...]


def _head(P, sa3, glob, clf, S):
    Cin = P.shape[-1]
    ws = []
    for lyr in sa3 + glob:
        ws += [jnp.transpose(lyr["W"]), lyr["gamma"].reshape(1, -1),
               lyr["beta"].reshape(1, -1)]
    ws += [jnp.transpose(clf["W"]), clf["b"].reshape(1, -1)]
    full = lambda s: pl.BlockSpec(s, lambda b: tuple(0 for _ in s))
    in_specs = [pl.BlockSpec((1, S, Cin), lambda b: (b, 0, 0))]
    in_specs += [full(w.shape) for w in ws]
    return pl.pallas_call(
        _head_body, grid=(_B,), in_specs=in_specs,
        out_specs=pl.BlockSpec((1, 1, 40), lambda b: (b, 0, 0)),
        out_shape=jax.ShapeDtypeStruct((_B, 1, 40), jnp.float32),
    )(P, *ws).reshape(_B, 40)


_SA1 = ((0.1, 16), (0.2, 32), (0.4, 128))
_SA2 = ((0.2, 32), (0.4, 64), (0.8, 128))


def _sa_msg_raw(pts, S, N, scale_cfg, scale_params):
    ox, oy, oz = _fps(pts, S)
    c = jnp.stack([ox, oy, oz], axis=-1)
    d2 = _pair_d2(c, pts, S, N)
    tab = jnp.pad(jnp.transpose(pts, (0, 2, 1)),
                  ((0, 0), (0, 0), (0, 125))).reshape(_B * N, 128)
    cpad = jnp.pad(c, ((0, 0), (0, 0), (0, 125)))
    feats = []
    for (radius, K), layers in zip(scale_cfg, scale_params):
        idx = _ballq_idx(d2, radius * radius, K, S, N).reshape(_B * S, K)
        g = _ball_gather(idx, tab, K, S)
        f = _mlp(g.reshape(_B, S, K, 128), cpad, layers, K, S, raw=True)
        feats.append(f)
    return c, jnp.concatenate(feats, axis=-1)


def _sa_msg_proj(c_prev, f_prev, S, N, scale_cfg, scale_params):
    pts = jnp.transpose(c_prev, (0, 2, 1))
    ox, oy, oz = _fps(pts, S)
    c = jnp.stack([ox, oy, oz], axis=-1)
    d2 = _pair_d2(c, pts, S, N)
    P = jnp.concatenate([c_prev, f_prev], axis=-1)
    us, vs = _proj(P, c, scale_params, N, S)
    feats = []
    for (radius, K), layers, u, v in zip(scale_cfg, scale_params, us, vs):
        idx = _ballq_idx(d2, radius * radius, K, S, N).reshape(_B * S, K)
        g = _ball_gather(idx, u.reshape(_B * N, 128), K, S)
        f = _mlp(g.reshape(_B, S, K, 128), v, layers, K, S, raw=False)
        feats.append(f)
    return c, jnp.concatenate(feats, axis=-1)


def kernel(points, params):
    c1, f1 = _sa_msg_raw(points[:, 0:3, :], 512, _N, _SA1, params["sa1"])
    c2, f2 = _sa_msg_proj(c1, f1, 128, 512, _SA2, params["sa2"])
    P3 = jnp.concatenate([c2, f2], axis=-1)
    cls = _head(P3, params["sa3"], params["mlp_global"], params["classifier"],
                128)
    kpi = jnp.zeros((_B, 1024), jnp.int32)
    return cls, kpi
